# baseline (speedup 1.0000x reference)
import jax
import jax.numpy as jnp
from jax.experimental import pallas as pl

_TILE = 2048


def _mlp_kernel(x_ref, w0a_ref, w0b_ref, b0_ref, w1_ref, b1_ref, out_ref):
    gus = x_ref[0]
    gis = x_ref[1]
    h = jnp.dot(gus, w0a_ref[...], preferred_element_type=jnp.float32)
    h += jnp.dot(gis, w0b_ref[...], preferred_element_type=jnp.float32)
    h = jnp.maximum(h + b0_ref[...], 0.0)
    out = jnp.sum(h * w1_ref[...], axis=1, keepdims=True) + b1_ref[...]
    out_ref[...] = out


def kernel(inputs, W0, b0, W1, b1):
    _, batch, k = inputs.shape
    h1 = W0.shape[1]
    w0a = W0[:k]
    w0b = W0[k:]
    b0r = b0.reshape(1, h1)
    w1r = W1.reshape(1, h1)
    b1r = b1.reshape(1, 1)
    tile = min(_TILE, batch)
    grid = (batch // tile,)
    return pl.pallas_call(
        _mlp_kernel,
        grid=grid,
        in_specs=[
            pl.BlockSpec((2, tile, k), lambda i: (0, i, 0)),
            pl.BlockSpec((k, h1), lambda i: (0, 0)),
            pl.BlockSpec((k, h1), lambda i: (0, 0)),
            pl.BlockSpec((1, h1), lambda i: (0, 0)),
            pl.BlockSpec((1, h1), lambda i: (0, 0)),
            pl.BlockSpec((1, 1), lambda i: (0, 0)),
        ],
        out_specs=pl.BlockSpec((tile, 1), lambda i: (i, 0)),
        out_shape=jax.ShapeDtypeStruct((batch, 1), jnp.float32),
    )(inputs, w0a, w0b, b0r, w1r, b1r)
